# Initial kernel scaffold; baseline (speedup 1.0000x reference)
import jax
import jax.numpy as jnp
from jax.experimental import pallas as pl


def kernel(x, edge_index, W_in, b_in, Ws0, bs0, Wd0, bd0, a0, Ws1, bs1, Wd1, bd1, a1, Ws2, bs2, Wd2, bd2, a2, W1, b1, W2, b2, W3, b3):
    raise NotImplementedError("write your pallas kernel here")


# speedup vs baseline: 39.2886x; 39.2886x over previous
import functools

import jax
import jax.numpy as jnp
from jax import lax
from jax.experimental import pallas as pl
from jax.experimental.pallas import tpu as pltpu
from jax.experimental.pallas import tpu_sc as plsc

_N = 10000
_E = 320000
_HID = 128
_HEADS = 8
_DH = 16
_SLOPE = 0.2

_ROWS = 1000
_GRID = _N // _ROWS
_CHUNK = 64
_NCHUNK = _E // _CHUNK
_NWORK = 32
_ROWS_PER_SUB = 624
_CPY = 48
_TAIL = _N - 16 * _ROWS_PER_SUB
_DROWS = 1280
_DPS = _DROWS // 16
_DCPY = 40


def _proj_body(x_ref, w_ref, b_ref, ws_ref, bs_ref, wd_ref, bd_ref,
               h_ref, fs_ref, fd_ref):
    h = jnp.dot(x_ref[...], w_ref[...], preferred_element_type=jnp.float32)
    h = h + b_ref[...]
    h_ref[...] = h
    fs_ref[...] = jnp.dot(h, ws_ref[...],
                          preferred_element_type=jnp.float32) + bs_ref[...]
    fd_ref[...] = jnp.dot(h, wd_ref[...],
                          preferred_element_type=jnp.float32) + bd_ref[...]


def _combine(num_ref, den_ref, hp_ref):
    num = num_ref[0] + num_ref[1]
    den = den_ref[0] + den_ref[1]
    recip = jnp.where(den > 0.0, 1.0 / den, 0.0)[:, :_HEADS]
    r = lax.broadcasted_iota(jnp.int32, (_HEADS, _HID), 0)
    c = lax.broadcasted_iota(jnp.int32, (_HEADS, _HID), 1) // _DH
    emat = (r == c).astype(jnp.float32)
    den128 = jnp.dot(recip, emat, preferred_element_type=jnp.float32)
    return jnp.maximum(num * den128 + hp_ref[...], 0.0)


def _combine_proj_body(num_ref, den_ref, hp_ref, ws_ref, bs_ref, wd_ref,
                       bd_ref, h_ref, fs_ref, fd_ref):
    h = _combine(num_ref, den_ref, hp_ref)
    h_ref[...] = h
    fs_ref[...] = jnp.dot(h, ws_ref[...],
                          preferred_element_type=jnp.float32) + bs_ref[...]
    fd_ref[...] = jnp.dot(h, wd_ref[...],
                          preferred_element_type=jnp.float32) + bd_ref[...]


def _combine_sum_body(num_ref, den_ref, hp_ref, out_ref):
    h = _combine(num_ref, den_ref, hp_ref)
    out_ref[...] = jnp.broadcast_to(jnp.sum(h, axis=0, keepdims=True),
                                    (8, _HID))


def _mlp_body(hs_ref, w1_ref, b1_ref, w2_ref, b2_ref, w3_ref, b3_ref,
              out_ref):
    hg = jnp.sum(hs_ref[...], axis=0, keepdims=True) * 0.125
    z = jnp.maximum(jnp.dot(hg, w1_ref[...],
                            preferred_element_type=jnp.float32) + b1_ref[...],
                    0.0)
    z = jnp.maximum(jnp.dot(z, w2_ref[...],
                            preferred_element_type=jnp.float32) + b2_ref[...],
                    0.0)
    o = jnp.dot(z, w3_ref[...],
                preferred_element_type=jnp.float32) + b3_ref[...]
    out_ref[...] = jnp.broadcast_to(o, (8, _HID))


_row_spec = pl.BlockSpec((_ROWS, _HID), lambda i: (i, 0))
_w_spec = pl.BlockSpec((_HID, _HID), lambda i: (0, 0))
_b_spec = pl.BlockSpec((1, _HID), lambda i: (0, 0))
_num_spec = pl.BlockSpec((2, _ROWS, _HID), lambda i: (0, i, 0))
_den_spec = pl.BlockSpec((2, _ROWS, _DH), lambda i: (0, i, 0))

_f32 = jnp.float32


def _tc_proj(x, w, b, ws, bs, wd, bd):
    return pl.pallas_call(
        _proj_body,
        grid=(_GRID,),
        in_specs=[_row_spec, _w_spec, _b_spec, _w_spec, _b_spec, _w_spec,
                  _b_spec],
        out_specs=[_row_spec, _row_spec, _row_spec],
        out_shape=[jax.ShapeDtypeStruct((_N, _HID), _f32)] * 3,
    )(x, w, b, ws, bs, wd, bd)


def _tc_combine_proj(num, den, hp, ws, bs, wd, bd):
    return pl.pallas_call(
        _combine_proj_body,
        grid=(_GRID,),
        in_specs=[_num_spec, _den_spec, _row_spec, _w_spec, _b_spec, _w_spec,
                  _b_spec],
        out_specs=[_row_spec, _row_spec, _row_spec],
        out_shape=[jax.ShapeDtypeStruct((_N, _HID), _f32)] * 3,
    )(num, den, hp, ws, bs, wd, bd)


def _tc_combine_sum(num, den, hp):
    return pl.pallas_call(
        _combine_sum_body,
        grid=(_GRID,),
        in_specs=[_num_spec, _den_spec, _row_spec],
        out_specs=pl.BlockSpec((8, _HID), lambda i: (i, 0)),
        out_shape=jax.ShapeDtypeStruct((_GRID * 8, _HID), _f32),
    )(num, den, hp)


def _tc_mlp(hs, w1, b1, w2, b2, w3, b3):
    return pl.pallas_call(
        _mlp_body,
        out_shape=jax.ShapeDtypeStruct((8, _HID), _f32),
    )(hs, w1, b1, w2, b2, w3, b3)


def _edge_body(fs_hbm, fd_hbm, src_hbm, dst_hbm, attn_hbm,
               num_hbm, den_hbm,
               el_v, er_v, exw_v, sidx_v, didx_v, didx2_v, attn_v,
               acc_sh, den_sh):
    cid = lax.axis_index("c")
    sid = lax.axis_index("s")
    wid = cid * 16 + sid

    pltpu.sync_copy(attn_hbm, attn_v)

    zero16 = jnp.zeros((16,), jnp.float32)

    def _zero_row(i, carry):
        for k in range(_HID // 16):
            el_v[i, pl.ds(k * 16, 16)] = zero16
            exw_v[i, pl.ds(k * 16, 16)] = zero16
        return carry

    lax.fori_loop(0, _CHUNK, _zero_row, 0)

    base = sid * _ROWS_PER_SUB
    for k in range(_ROWS_PER_SUB // _CPY):
        rows = pl.ds(base + k * _CPY, _CPY)
        pltpu.sync_copy(el_v.at[pl.ds(0, _CPY), :], acc_sh.at[rows, :])

    dbase = sid * _DPS
    for k in range(_DPS // _DCPY):
        rows = pl.ds(dbase + k * _DCPY, _DCPY)
        pltpu.sync_copy(el_v.at[pl.ds(0, _DCPY), :], den_sh.at[rows, :])

    @pl.when(sid == 15)
    def _zero_tail():
        rows = pl.ds(16 * _ROWS_PER_SUB, _TAIL)
        pltpu.sync_copy(el_v.at[pl.ds(0, _TAIL), :], acc_sh.at[rows, :])

    plsc.subcore_barrier()

    ah = [attn_v[pl.ds(k * 16, 16)] for k in range(_HEADS)]
    lane = lax.iota(jnp.int32, 16)
    lo_mask = lane < _HEADS
    perms = [jnp.bitwise_xor(lane, w).reshape(16, 1) for w in (8, 4, 2, 1)]
    _dnums = lax.GatherDimensionNumbers(offset_dims=(),
                                        collapsed_slice_dims=(0,),
                                        start_index_map=(0,))

    def _shuf(v, p):
        return lax.gather(v, p, _dnums, (1,),
                          mode=lax.GatherScatterMode.PROMISE_IN_BOUNDS)

    def _allsum(v):
        for p in perms:
            v = v + _shuf(v, p)
        return v


    def _chunk(t, carry):
        ci = wid + t * _NWORK
        off = ci * _CHUNK
        pltpu.sync_copy(src_hbm.at[pl.ds(off, _CHUNK)], sidx_v)
        pltpu.sync_copy(dst_hbm.at[pl.ds(off, _CHUNK)], didx_v)
        pltpu.sync_copy(fs_hbm.at[sidx_v], el_v)
        pltpu.sync_copy(fd_hbm.at[didx_v], er_v)

        def _group(g, c2):
            goff = g * 16
            dvec = didx_v[pl.ds(goff, 16)]
            didx2_v[pl.ds(goff, 16)] = lax.shift_right_logical(dvec, 3)
            for j2 in range(16):
                j = goff + j2
                exacc = jnp.zeros((16,), jnp.float32)
                for hh in range(_HEADS):
                    elh = el_v[j, pl.ds(hh * 16, 16)]
                    erh = er_v[j, pl.ds(hh * 16, 16)]
                    tt = elh + erh
                    tt = jnp.maximum(tt, tt * _SLOPE)
                    exv = jnp.exp(_allsum(tt * ah[hh]))
                    el_v[j, pl.ds(hh * 16, 16)] = elh * exv
                    exacc = jnp.where(lane == hh, exv, exacc)
                for k in range(_HID // 16):
                    exw_v[j, pl.ds(k * 16, 16)] = zero16
                slot = (dvec[j2] & 7) * 16
                exw_v[j, pl.ds(slot, 16)] = exacc
            return c2

        lax.fori_loop(0, _CHUNK // 16, _group, 0)

        pltpu.sync_copy(el_v, acc_sh.at[didx_v], add=True)
        pltpu.sync_copy(exw_v, den_sh.at[didx2_v], add=True)
        return carry

    nfull = _NCHUNK // _NWORK
    nrem = _NCHUNK - nfull * _NWORK
    ntrips = jnp.where(wid < nrem, nfull + 1, nfull)
    lax.fori_loop(0, ntrips, _chunk, 0)

    plsc.subcore_barrier()

    for k in range(_ROWS_PER_SUB // _CPY):
        rows = pl.ds(base + k * _CPY, _CPY)
        pltpu.sync_copy(acc_sh.at[rows, :], el_v.at[pl.ds(0, _CPY), :])
        pltpu.sync_copy(el_v.at[pl.ds(0, _CPY), :], num_hbm.at[cid, rows, :])

    for k in range(_DPS // _DCPY):
        rows = pl.ds(dbase + k * _DCPY, _DCPY)
        pltpu.sync_copy(den_sh.at[rows, :], er_v.at[pl.ds(0, _DCPY), :])
        pltpu.sync_copy(er_v.at[pl.ds(0, _DCPY), :], den_hbm.at[cid, rows, :])

    @pl.when(sid == 15)
    def _out_tail():
        rows = pl.ds(16 * _ROWS_PER_SUB, _TAIL)
        pltpu.sync_copy(acc_sh.at[rows, :], el_v.at[pl.ds(0, _TAIL), :])
        pltpu.sync_copy(el_v.at[pl.ds(0, _TAIL), :], num_hbm.at[cid, rows, :])


def _sc_edge_pass(fs, fd, src, dst, attn_flat):
    mesh = plsc.VectorSubcoreMesh(core_axis_name="c", subcore_axis_name="s")
    fn = functools.partial(
        pl.kernel,
        mesh=mesh,
        out_type=(jax.ShapeDtypeStruct((2, _N, _HID), _f32),
                  jax.ShapeDtypeStruct((2, _DROWS, _HID), _f32)),
        scratch_types=[
            pltpu.VMEM((_CHUNK, _HID), _f32),
            pltpu.VMEM((_CHUNK, _HID), _f32),
            pltpu.VMEM((_CHUNK, _HID), _f32),
            pltpu.VMEM((_CHUNK,), jnp.int32),
            pltpu.VMEM((_CHUNK,), jnp.int32),
            pltpu.VMEM((_CHUNK,), jnp.int32),
            pltpu.VMEM((_HID,), _f32),
            pltpu.VMEM_SHARED((_N, _HID), _f32),
            pltpu.VMEM_SHARED((_DROWS, _HID), _f32),
        ],
    )(_edge_body)
    num, denw = fn(fs, fd, src, dst, attn_flat)
    den = denw.reshape(2, _DROWS * 8, _DH)[:, :_N, :]
    return num, den


def kernel(x, edge_index, W_in, b_in, Ws0, bs0, Wd0, bd0, a0, Ws1, bs1, Wd1,
           bd1, a1, Ws2, bs2, Wd2, bd2, a2, W1, b1, W2, b2, W3, b3):
    src = edge_index[0]
    dst = edge_index[1]
    b_in2 = b_in.reshape(1, _HID)

    layers = [(Ws0, bs0, Wd0, bd0, a0), (Ws1, bs1, Wd1, bd1, a1),
              (Ws2, bs2, Wd2, bd2, a2)]

    h, fs, fd = _tc_proj(x, W_in, b_in2, layers[0][0],
                         layers[0][1].reshape(1, _HID), layers[0][2],
                         layers[0][3].reshape(1, _HID))

    for l in range(3):
        attn_flat = layers[l][4].reshape(_HID)
        num, den = _sc_edge_pass(fs, fd, src, dst, attn_flat)
        if l < 2:
            nxt = layers[l + 1]
            h, fs, fd = _tc_combine_proj(num, den, h, nxt[0],
                                         nxt[1].reshape(1, _HID), nxt[2],
                                         nxt[3].reshape(1, _HID))
        else:
            hs = _tc_combine_sum(num, den, h)

    w2p = jnp.pad(W2, ((0, 0), (0, 64)))
    b2p = jnp.pad(b2, (0, 64)).reshape(1, _HID)
    w3p = jnp.pad(W3, ((0, 64), (0, 118)))
    b3p = jnp.pad(b3, (0, 118)).reshape(1, _HID)
    out = _tc_mlp(hs, W1, b1.reshape(1, _HID), w2p, b2p, w3p, b3p)
    return out[0:1, 0:10]


# speedup vs baseline: 52.2845x; 1.3308x over previous
import functools

import jax
import jax.numpy as jnp
from jax import lax
from jax.experimental import pallas as pl
from jax.experimental.pallas import tpu as pltpu
from jax.experimental.pallas import tpu_sc as plsc

_N = 10000
_E = 320000
_HID = 128
_HEADS = 8
_DH = 16
_SLOPE = 0.2

_ROWS = 1000
_GRID = _N // _ROWS
_CHUNK = 64
_NCHUNK = _E // _CHUNK
_NWORK = 32
_ROWS_PER_SUB = 624
_CPY = 48
_TAIL = _N - 16 * _ROWS_PER_SUB
_DROWS = 1280
_DPS = _DROWS // 16
_DCPY = 40


def _proj_body(x_ref, w_ref, b_ref, ws_ref, bs_ref, wd_ref, bd_ref,
               h_ref, fs_ref, fd_ref):
    h = jnp.dot(x_ref[...], w_ref[...], preferred_element_type=jnp.float32)
    h = h + b_ref[...]
    h_ref[...] = h
    fs_ref[...] = jnp.dot(h, ws_ref[...],
                          preferred_element_type=jnp.float32) + bs_ref[...]
    fd_ref[...] = jnp.dot(h, wd_ref[...],
                          preferred_element_type=jnp.float32) + bd_ref[...]


def _combine(num_ref, den_ref, hp_ref):
    num = num_ref[0] + num_ref[1]
    den = den_ref[0] + den_ref[1]
    recip = jnp.where(den > 0.0, 1.0 / den, 0.0)[:, :_HEADS]
    r = lax.broadcasted_iota(jnp.int32, (_HEADS, _HID), 0)
    c = lax.broadcasted_iota(jnp.int32, (_HEADS, _HID), 1) // _DH
    emat = (r == c).astype(jnp.float32)
    den128 = jnp.dot(recip, emat, preferred_element_type=jnp.float32)
    return jnp.maximum(num * den128 + hp_ref[...], 0.0)


def _combine_proj_body(num_ref, den_ref, hp_ref, ws_ref, bs_ref, wd_ref,
                       bd_ref, h_ref, fs_ref, fd_ref):
    h = _combine(num_ref, den_ref, hp_ref)
    h_ref[...] = h
    fs_ref[...] = jnp.dot(h, ws_ref[...],
                          preferred_element_type=jnp.float32) + bs_ref[...]
    fd_ref[...] = jnp.dot(h, wd_ref[...],
                          preferred_element_type=jnp.float32) + bd_ref[...]


def _combine_sum_body(num_ref, den_ref, hp_ref, out_ref):
    h = _combine(num_ref, den_ref, hp_ref)
    out_ref[...] = jnp.broadcast_to(jnp.sum(h, axis=0, keepdims=True),
                                    (8, _HID))


def _mlp_body(hs_ref, w1_ref, b1_ref, w2_ref, b2_ref, w3_ref, b3_ref,
              out_ref):
    hg = jnp.sum(hs_ref[...], axis=0, keepdims=True) * 0.125
    z = jnp.maximum(jnp.dot(hg, w1_ref[...],
                            preferred_element_type=jnp.float32) + b1_ref[...],
                    0.0)
    z = jnp.maximum(jnp.dot(z, w2_ref[...],
                            preferred_element_type=jnp.float32) + b2_ref[...],
                    0.0)
    o = jnp.dot(z, w3_ref[...],
                preferred_element_type=jnp.float32) + b3_ref[...]
    out_ref[...] = jnp.broadcast_to(o, (8, _HID))


_row_spec = pl.BlockSpec((_ROWS, _HID), lambda i: (i, 0))
_w_spec = pl.BlockSpec((_HID, _HID), lambda i: (0, 0))
_b_spec = pl.BlockSpec((1, _HID), lambda i: (0, 0))
_num_spec = pl.BlockSpec((2, _ROWS, _HID), lambda i: (0, i, 0))
_den_spec = pl.BlockSpec((2, _ROWS, _DH), lambda i: (0, i, 0))

_f32 = jnp.float32


def _tc_proj(x, w, b, ws, bs, wd, bd):
    return pl.pallas_call(
        _proj_body,
        grid=(_GRID,),
        in_specs=[_row_spec, _w_spec, _b_spec, _w_spec, _b_spec, _w_spec,
                  _b_spec],
        out_specs=[_row_spec, _row_spec, _row_spec],
        out_shape=[jax.ShapeDtypeStruct((_N, _HID), _f32)] * 3,
    )(x, w, b, ws, bs, wd, bd)


def _tc_combine_proj(num, den, hp, ws, bs, wd, bd):
    return pl.pallas_call(
        _combine_proj_body,
        grid=(_GRID,),
        in_specs=[_num_spec, _den_spec, _row_spec, _w_spec, _b_spec, _w_spec,
                  _b_spec],
        out_specs=[_row_spec, _row_spec, _row_spec],
        out_shape=[jax.ShapeDtypeStruct((_N, _HID), _f32)] * 3,
    )(num, den, hp, ws, bs, wd, bd)


def _tc_combine_sum(num, den, hp):
    return pl.pallas_call(
        _combine_sum_body,
        grid=(_GRID,),
        in_specs=[_num_spec, _den_spec, _row_spec],
        out_specs=pl.BlockSpec((8, _HID), lambda i: (i, 0)),
        out_shape=jax.ShapeDtypeStruct((_GRID * 8, _HID), _f32),
    )(num, den, hp)


def _tc_mlp(hs, w1, b1, w2, b2, w3, b3):
    return pl.pallas_call(
        _mlp_body,
        out_shape=jax.ShapeDtypeStruct((8, _HID), _f32),
    )(hs, w1, b1, w2, b2, w3, b3)


def _edge_body(fs_hbm, fd_hbm, src_hbm, dst_hbm, attn_hbm,
               num_hbm, den_hbm,
               el0_v, el1_v, er_v, exw_v, sidx0_v, sidx1_v,
               didx0_v, didx1_v, didx2_v, attn_v, acc_sh, den_sh,
               gel0, gel1, ger):
    el_b = (el0_v, el1_v)
    sidx_b = (sidx0_v, sidx1_v)
    didx_b = (didx0_v, didx1_v)
    gel_b = (gel0, gel1)
    el_v = el0_v
    cid = lax.axis_index("c")
    sid = lax.axis_index("s")
    wid = cid * 16 + sid

    pltpu.sync_copy(attn_hbm, attn_v)

    zero16 = jnp.zeros((16,), jnp.float32)

    def _zero_row(i, carry):
        for k in range(_HID // 16):
            el_v[i, pl.ds(k * 16, 16)] = zero16
            exw_v[i, pl.ds(k * 16, 16)] = zero16
        return carry

    lax.fori_loop(0, _CHUNK, _zero_row, 0)

    base = sid * _ROWS_PER_SUB
    for k in range(_ROWS_PER_SUB // _CPY):
        rows = pl.ds(base + k * _CPY, _CPY)
        pltpu.sync_copy(el_v.at[pl.ds(0, _CPY), :], acc_sh.at[rows, :])

    dbase = sid * _DPS
    for k in range(_DPS // _DCPY):
        rows = pl.ds(dbase + k * _DCPY, _DCPY)
        pltpu.sync_copy(el_v.at[pl.ds(0, _DCPY), :], den_sh.at[rows, :])

    @pl.when(sid == 15)
    def _zero_tail():
        rows = pl.ds(16 * _ROWS_PER_SUB, _TAIL)
        pltpu.sync_copy(el_v.at[pl.ds(0, _TAIL), :], acc_sh.at[rows, :])

    plsc.subcore_barrier()

    ah = [attn_v[pl.ds(k * 16, 16)] for k in range(_HEADS)]
    lane = lax.iota(jnp.int32, 16)
    p8, p4, p2, p1 = [jnp.bitwise_xor(lane, w).reshape(16, 1)
                      for w in (8, 4, 2, 1)]
    m8 = lane < 8
    m4 = (lane & 4) == 0
    m2 = (lane & 2) == 0
    _lanepos = (0, 8, 4, 12, 2, 10, 6, 14)
    bcp = [jnp.full((16, 1), _lanepos[h], jnp.int32) for h in range(_HEADS)]
    _rev = (((lane & 1) << 2) | (lane & 2) | ((lane >> 2) & 1)) * 2
    packp = jnp.where(m8, _rev, 0).reshape(16, 1)
    _dnums = lax.GatherDimensionNumbers(offset_dims=(),
                                        collapsed_slice_dims=(0,),
                                        start_index_map=(0,))

    def _shuf(v, p):
        return lax.gather(v, p, _dnums, (1,),
                          mode=lax.GatherScatterMode.PROMISE_IN_BOUNDS)

    def _logits8(ts):
        u = [ts[h] + _shuf(ts[h], p8) for h in range(8)]
        m = [jnp.where(m8, u[2 * p], u[2 * p + 1]) for p in range(4)]
        w = [m[p] + _shuf(m[p], p4) for p in range(4)]
        n = [jnp.where(m4, w[0], w[1]), jnp.where(m4, w[2], w[3])]
        y = [n[p] + _shuf(n[p], p2) for p in range(2)]
        z = jnp.where(m2, y[0], y[1])
        return z + _shuf(z, p1)


    def _load_idx(t, b):
        off = (wid + t * _NWORK) * _CHUNK
        pltpu.sync_copy(src_hbm.at[pl.ds(off, _CHUNK)], sidx_b[b])
        pltpu.sync_copy(dst_hbm.at[pl.ds(off, _CHUNK)], didx_b[b])

    def _compute(b):
        elv = el_b[b]
        didxv = didx_b[b]

        def _group(g, c2):
            goff = g * 16
            dvec = didxv[pl.ds(goff, 16)]
            didx2_v[pl.ds(goff, 16)] = lax.shift_right_logical(dvec, 3)
            for j2 in range(16):
                j = goff + j2
                ts = []
                for hh in range(_HEADS):
                    tt = elv[j, pl.ds(hh * 16, 16)] + \
                        er_v[j, pl.ds(hh * 16, 16)]
                    tt = jnp.maximum(tt, tt * _SLOPE)
                    ts.append(tt * ah[hh])
                ex16 = jnp.exp(_logits8(ts))
                for hh in range(_HEADS):
                    sl = pl.ds(hh * 16, 16)
                    elv[j, sl] = elv[j, sl] * _shuf(ex16, bcp[hh])
                for k in range(_HID // 16):
                    exw_v[j, pl.ds(k * 16, 16)] = zero16
                slot = (dvec[j2] & 7) * 16
                exw_v[j, pl.ds(slot, 16)] = _shuf(ex16, packp)
            return c2

        lax.fori_loop(0, _CHUNK // 16, _group, 0)

    _compute_pre = _compute

    nfull = _NCHUNK // _NWORK
    nrem = _NCHUNK - nfull * _NWORK
    ntrips = jnp.where(wid < nrem, nfull + 1, nfull)

    _load_idx(0, 0)
    pltpu.async_copy(fs_hbm.at[sidx0_v], el0_v, gel0)
    pltpu.async_copy(fd_hbm.at[didx0_v], er_v, ger)

    def _iter(t, b):
        nb = 1 - b
        elv = el_b[b]

        @pl.when(t + 1 < ntrips)
        def _pf():
            _load_idx(t + 1, nb)

        pltpu.make_async_copy(fs_hbm.at[sidx_b[b]], elv, gel_b[b]).wait()
        pltpu.make_async_copy(fd_hbm.at[didx_b[b]], er_v, ger).wait()

        @pl.when(t + 1 < ntrips)
        def _go_el():
            pltpu.async_copy(fs_hbm.at[sidx_b[nb]], el_b[nb], gel_b[nb])

        _compute_pre(b)

        @pl.when(t + 1 < ntrips)
        def _go_er():
            pltpu.async_copy(fd_hbm.at[didx_b[nb]], er_v, ger)

        pltpu.sync_copy(elv, acc_sh.at[didx_b[b]], add=True)
        pltpu.sync_copy(exw_v, den_sh.at[didx2_v], add=True)

    def _pair(u, carry):
        t0 = u * 2
        _iter(t0, 0)

        @pl.when(t0 + 1 < ntrips)
        def _odd():
            _iter(t0 + 1, 1)

        return carry

    lax.fori_loop(0, (ntrips + 1) // 2, _pair, 0)

    plsc.subcore_barrier()

    for k in range(_ROWS_PER_SUB // _CPY):
        rows = pl.ds(base + k * _CPY, _CPY)
        pltpu.sync_copy(acc_sh.at[rows, :], el_v.at[pl.ds(0, _CPY), :])
        pltpu.sync_copy(el_v.at[pl.ds(0, _CPY), :], num_hbm.at[cid, rows, :])

    for k in range(_DPS // _DCPY):
        rows = pl.ds(dbase + k * _DCPY, _DCPY)
        pltpu.sync_copy(den_sh.at[rows, :], er_v.at[pl.ds(0, _DCPY), :])
        pltpu.sync_copy(er_v.at[pl.ds(0, _DCPY), :], den_hbm.at[cid, rows, :])

    @pl.when(sid == 15)
    def _out_tail():
        rows = pl.ds(16 * _ROWS_PER_SUB, _TAIL)
        pltpu.sync_copy(acc_sh.at[rows, :], el_v.at[pl.ds(0, _TAIL), :])
        pltpu.sync_copy(el_v.at[pl.ds(0, _TAIL), :], num_hbm.at[cid, rows, :])


def _sc_edge_pass(fs, fd, src, dst, attn_flat):
    mesh = plsc.VectorSubcoreMesh(core_axis_name="c", subcore_axis_name="s")
    fn = functools.partial(
        pl.kernel,
        mesh=mesh,
        out_type=(jax.ShapeDtypeStruct((2, _N, _HID), _f32),
                  jax.ShapeDtypeStruct((2, _DROWS, _HID), _f32)),
        scratch_types=[
            pltpu.VMEM((_CHUNK, _HID), _f32),
            pltpu.VMEM((_CHUNK, _HID), _f32),
            pltpu.VMEM((_CHUNK, _HID), _f32),
            pltpu.VMEM((_CHUNK, _HID), _f32),
            pltpu.VMEM((_CHUNK,), jnp.int32),
            pltpu.VMEM((_CHUNK,), jnp.int32),
            pltpu.VMEM((_CHUNK,), jnp.int32),
            pltpu.VMEM((_CHUNK,), jnp.int32),
            pltpu.VMEM((_CHUNK,), jnp.int32),
            pltpu.VMEM((_HID,), _f32),
            pltpu.VMEM_SHARED((_N, _HID), _f32),
            pltpu.VMEM_SHARED((_DROWS, _HID), _f32),
            pltpu.SemaphoreType.DMA,
            pltpu.SemaphoreType.DMA,
            pltpu.SemaphoreType.DMA,
        ],
    )(_edge_body)
    num, denw = fn(fs, fd, src, dst, attn_flat)
    den = denw.reshape(2, _DROWS * 8, _DH)[:, :_N, :]
    return num, den


def kernel(x, edge_index, W_in, b_in, Ws0, bs0, Wd0, bd0, a0, Ws1, bs1, Wd1,
           bd1, a1, Ws2, bs2, Wd2, bd2, a2, W1, b1, W2, b2, W3, b3):
    src = edge_index[0]
    dst = edge_index[1]
    b_in2 = b_in.reshape(1, _HID)

    layers = [(Ws0, bs0, Wd0, bd0, a0), (Ws1, bs1, Wd1, bd1, a1),
              (Ws2, bs2, Wd2, bd2, a2)]

    h, fs, fd = _tc_proj(x, W_in, b_in2, layers[0][0],
                         layers[0][1].reshape(1, _HID), layers[0][2],
                         layers[0][3].reshape(1, _HID))

    for l in range(3):
        attn_flat = layers[l][4].reshape(_HID)
        num, den = _sc_edge_pass(fs, fd, src, dst, attn_flat)
        if l < 2:
            nxt = layers[l + 1]
            h, fs, fd = _tc_combine_proj(num, den, h, nxt[0],
                                         nxt[1].reshape(1, _HID), nxt[2],
                                         nxt[3].reshape(1, _HID))
        else:
            hs = _tc_combine_sum(num, den, h)

    w2p = jnp.pad(W2, ((0, 0), (0, 64)))
    b2p = jnp.pad(b2, (0, 64)).reshape(1, _HID)
    w3p = jnp.pad(W3, ((0, 64), (0, 118)))
    b3p = jnp.pad(b3, (0, 118)).reshape(1, _HID)
    out = _tc_mlp(hs, W1, b1.reshape(1, _HID), w2p, b2p, w3p, b3p)
    return out[0:1, 0:10]
